# baseline (speedup 1.0000x reference)
import functools

import jax
import jax.numpy as jnp
import numpy as np
from jax import lax
from jax.experimental import pallas as pl
from jax.experimental.pallas import tpu as pltpu
from jax.experimental.pallas import tpu_sc as plsc

_CHUNK = 640
_NSLOTS = 4
_SPLIT = 1


@functools.partial(jax.jit, static_argnames=("nb", "nc", "ns", "bpw", "nrounds"))
def _sc_gather(flat_idx, table4, *, nb, nc, ns, bpw, nrounds):
    SP = _SPLIT
    D4 = table4.shape[1]
    C = _CHUNK
    N = _NSLOTS
    mesh = plsc.VectorSubcoreMesh(core_axis_name="c", subcore_axis_name="s")
    rows_per_grp = 16 // SP
    sp_shift = SP.bit_length() - 1

    @functools.partial(
        pl.kernel,
        mesh=mesh,
        out_type=jax.ShapeDtypeStruct((nb * SP, D4), jnp.float32),
        scratch_types=[
            pltpu.VMEM((bpw,), jnp.int32),
            [pltpu.VMEM((C * SP, D4), jnp.float32) for _ in range(N)],
            [pltpu.SemaphoreType.DMA for _ in range(N)],
            [pltpu.SemaphoreType.DMA for _ in range(N)],
        ],
        compiler_params=pltpu.CompilerParams(
            use_tc_tiling_on_sc=False, needs_layout_passes=False
        ),
    )
    def k(idx_hbm, table_hbm, out_hbm, idx_all, rows, sg, sw):
        wid = lax.axis_index("s") * nc + lax.axis_index("c")
        base = wid * bpw
        lanes = lax.iota(jnp.int32, 16)
        rep_const = lax.shift_right_logical(lanes, sp_shift)
        off_const = lax.bitwise_and(lanes, SP - 1)
        pltpu.sync_copy(idx_hbm.at[pl.ds(base, bpw)], idx_all)

        def gather(j, buf, sem):
            def gbody(g, carry):
                rbase = j * C + g * rows_per_grp
                vals = plsc.load_gather(idx_all, [rbase + rep_const])
                ov = vals * SP + off_const
                pltpu.async_copy(table_hbm.at[ov], buf.at[pl.ds(g * 16, 16)], sem)
                return carry

            lax.fori_loop(0, C // rows_per_grp, gbody, 0, unroll=8)

        def wait_gather(buf, sem):
            pltpu.make_async_copy(out_hbm.at[pl.ds(0, C * SP)], buf, sem).wait()

        def write(j, buf, sem):
            pltpu.async_copy(buf, out_hbm.at[pl.ds((base + j * C) * SP, C * SP)], sem)

        def wait_write(j, buf, sem):
            pltpu.make_async_copy(
                buf, out_hbm.at[pl.ds((base + j * C) * SP, C * SP)], sem
            ).wait()

        for s in range(N):
            gather(s, rows[s], sg[s])

        def body(t, carry):
            for s in range(N):
                j = N * t + s
                wait_gather(rows[s], sg[s])
                write(j, rows[s], sw[s])

                @pl.when(t < nrounds - 1)
                def _():
                    wait_write(j, rows[s], sw[s])
                    gather(j + N, rows[s], sg[s])

            return carry

        lax.fori_loop(0, nrounds, body, 0, unroll=False)
        for s in range(N):
            wait_write(N * (nrounds - 1) + s, rows[s], sw[s])

    return k(flat_idx, table4)


def kernel(input_tokens, table):
    B, H = input_tokens.shape
    V, D = table.shape
    info = plsc.get_sparse_core_info()
    nc, ns = info.num_cores, info.num_subcores
    nb = B * H
    nw = nc * ns
    bpw = nb // nw
    nrounds = bpw // (_NSLOTS * _CHUNK)
    flat = input_tokens.reshape(nb).astype(jnp.int32)
    table4 = table.reshape(V * _SPLIT, D // _SPLIT)
    out = _sc_gather(flat, table4, nb=nb, nc=nc, ns=ns, bpw=bpw, nrounds=nrounds)
    return out.reshape(B, H, D)


# speedup vs baseline: 1.6188x; 1.6188x over previous
import functools

import jax
import jax.numpy as jnp
import numpy as np
from jax import lax
from jax.experimental import pallas as pl
from jax.experimental.pallas import tpu as pltpu
from jax.experimental.pallas import tpu_sc as plsc

_CHUNK = 640
_NSLOTS = 4
_SPLIT = 2


@functools.partial(jax.jit, static_argnames=("nb", "nc", "ns", "bpw", "nrounds"))
def _sc_gather(flat_idx, table4, *, nb, nc, ns, bpw, nrounds):
    SP = _SPLIT
    D4 = table4.shape[1]
    C = _CHUNK
    N = _NSLOTS
    mesh = plsc.VectorSubcoreMesh(core_axis_name="c", subcore_axis_name="s")
    rows_per_grp = 16 // SP
    sp_shift = SP.bit_length() - 1

    @functools.partial(
        pl.kernel,
        mesh=mesh,
        out_type=jax.ShapeDtypeStruct((nb * SP, D4), jnp.float32),
        scratch_types=[
            pltpu.VMEM((bpw,), jnp.int32),
            [pltpu.VMEM((C * SP, D4), jnp.float32) for _ in range(N)],
            [pltpu.SemaphoreType.DMA for _ in range(N)],
            [pltpu.SemaphoreType.DMA for _ in range(N)],
        ],
        compiler_params=pltpu.CompilerParams(
            use_tc_tiling_on_sc=False, needs_layout_passes=False
        ),
    )
    def k(idx_hbm, table_hbm, out_hbm, idx_all, rows, sg, sw):
        wid = lax.axis_index("s") * nc + lax.axis_index("c")
        base = wid * bpw
        lanes = lax.iota(jnp.int32, 16)
        rep_const = lax.shift_right_logical(lanes, sp_shift)
        off_const = lax.bitwise_and(lanes, SP - 1)
        pltpu.sync_copy(idx_hbm.at[pl.ds(base, bpw)], idx_all)

        def gather(j, buf, sem):
            def gbody(g, carry):
                rbase = j * C + g * rows_per_grp
                vals = plsc.load_gather(idx_all, [rbase + rep_const])
                ov = vals * SP + off_const
                pltpu.async_copy(table_hbm.at[ov], buf.at[pl.ds(g * 16, 16)], sem)
                return carry

            lax.fori_loop(0, C // rows_per_grp, gbody, 0, unroll=8)

        def wait_gather(buf, sem):
            pltpu.make_async_copy(out_hbm.at[pl.ds(0, C * SP)], buf, sem).wait()

        def write(j, buf, sem):
            pltpu.async_copy(buf, out_hbm.at[pl.ds((base + j * C) * SP, C * SP)], sem)

        def wait_write(j, buf, sem):
            pltpu.make_async_copy(
                buf, out_hbm.at[pl.ds((base + j * C) * SP, C * SP)], sem
            ).wait()

        for s in range(N):
            gather(s, rows[s], sg[s])

        def body(t, carry):
            for s in range(N):
                j = N * t + s
                wait_gather(rows[s], sg[s])
                write(j, rows[s], sw[s])

                @pl.when(t < nrounds - 1)
                def _():
                    wait_write(j, rows[s], sw[s])
                    gather(j + N, rows[s], sg[s])

            return carry

        lax.fori_loop(0, nrounds, body, 0, unroll=False)
        for s in range(N):
            wait_write(N * (nrounds - 1) + s, rows[s], sw[s])

    return k(flat_idx, table4)


def kernel(input_tokens, table):
    B, H = input_tokens.shape
    V, D = table.shape
    info = plsc.get_sparse_core_info()
    nc, ns = info.num_cores, info.num_subcores
    nb = B * H
    nw = nc * ns
    bpw = nb // nw
    nrounds = bpw // (_NSLOTS * _CHUNK)
    flat = input_tokens.reshape(nb).astype(jnp.int32)
    table4 = table.reshape(V * _SPLIT, D // _SPLIT)
    out = _sc_gather(flat, table4, nb=nb, nc=nc, ns=ns, bpw=bpw, nrounds=nrounds)
    return out.reshape(B, H, D)
